# baseline (speedup 1.0000x reference)
import functools

import jax
import jax.numpy as jnp
from jax import lax
from jax.experimental import pallas as pl
from jax.experimental.pallas import tpu as pltpu
from jax.experimental.pallas import tpu_sc as plsc

_N = 10000
_E = 160000


def _tc_layer_body(*refs, relu_in, dh, nx):
    x_refs = refs[:nx]
    norm_ref, w_ref, out0_ref, out1_ref = refs[nx:]
    if nx == 1:
        x = x_refs[0][...]
    else:
        x = jnp.concatenate([r[...] for r in x_refs], axis=1)
    nrm = norm_ref[...]
    if relu_in:
        x = jnp.maximum(x * nrm, 0.0)
    y = jnp.dot(x, w_ref[...], preferred_element_type=jnp.float32)
    y = y * nrm
    out0_ref[...] = y[:, :dh]
    out1_ref[...] = y[:, dh:]


def _tc_layer(xs, norm, w, relu_in):
    n = xs[0].shape[0]
    dout = w.shape[1]
    dh = dout // 2
    blk = 2000
    return pl.pallas_call(
        functools.partial(_tc_layer_body, relu_in=relu_in, dh=dh, nx=len(xs)),
        grid=(n // blk,),
        in_specs=[
            pl.BlockSpec((blk, x.shape[1]), lambda i: (i, 0)) for x in xs
        ] + [
            pl.BlockSpec((blk, 1), lambda i: (i, 0)),
            pl.BlockSpec(w.shape, lambda i: (0, 0)),
        ],
        out_specs=[
            pl.BlockSpec((blk, dh), lambda i: (i, 0)),
            pl.BlockSpec((blk, dh), lambda i: (i, 0)),
        ],
        out_shape=[
            jax.ShapeDtypeStruct((n, dh), jnp.float32),
            jax.ShapeDtypeStruct((n, dh), jnp.float32),
        ],
    )(*xs, norm, w)


def _scale_body(x0_ref, x1_ref, norm_ref, o_ref):
    x = jnp.concatenate([x0_ref[...], x1_ref[...]], axis=1)
    o_ref[...] = x * norm_ref[...]


def _final_scale(x0, x1, norm):
    n, dh = x0.shape
    blk = 2000
    return pl.pallas_call(
        _scale_body,
        grid=(n // blk,),
        in_specs=[
            pl.BlockSpec((blk, dh), lambda i: (i, 0)),
            pl.BlockSpec((blk, dh), lambda i: (i, 0)),
            pl.BlockSpec((blk, 1), lambda i: (i, 0)),
        ],
        out_specs=pl.BlockSpec((blk, 2 * dh), lambda i: (i, 0)),
        out_shape=jax.ShapeDtypeStruct((n, 2 * dh), jnp.float32),
    )(x0, x1, norm)


_K = 80
_NSTG = 5
_CPS = 25


def _make_sc_agg(d2):
    rpt = 624

    mesh = plsc.VectorSubcoreMesh(core_axis_name="c", subcore_axis_name="s")

    @functools.partial(
        pl.kernel,
        mesh=mesh,
        compiler_params=pltpu.CompilerParams(use_tc_tiling_on_sc=(d2 % 128 == 0)),
        out_type=[
            jax.ShapeDtypeStruct((_N, d2), jnp.float32),
            jax.ShapeDtypeStruct((_N, d2), jnp.float32),
        ],
        scratch_types=[
            pltpu.VMEM((2, _CPS, _K), jnp.int32),
            pltpu.VMEM((2, _CPS, _K), jnp.int32),
            pltpu.VMEM((2, _K, d2), jnp.float32),
            pltpu.VMEM_SHARED((_N, d2), jnp.float32),
            pltpu.SemaphoreType.DMA,
            pltpu.SemaphoreType.DMA,
            pltpu.SemaphoreType.DMA,
            pltpu.SemaphoreType.DMA,
        ],
    )
    def agg(ei_hbm, g0_hbm, g1_hbm, zero_hbm, out0_hbm, out1_hbm,
            src_v, dst_v, rows_v, acc_sh, gsem0, gsem1, isem0, isem1):
        c = lax.axis_index("c")
        s = lax.axis_index("s")
        row0 = s * rpt
        tail = 16 * rpt

        pltpu.sync_copy(zero_hbm, acc_sh.at[pl.ds(row0, rpt)])

        @pl.when(s == 15)
        def _():
            pltpu.sync_copy(zero_hbm.at[pl.ds(0, 16)],
                            acc_sh.at[pl.ds(tail, 16)])

        plsc.subcore_barrier()

        def run(g_hbm, out_hbm):
            gsems = (gsem0, gsem1)
            isems = (isem0, isem1)

            def istart(t):
                ib = t % 2
                pltpu.async_copy(ei_hbm.at[0, s, t], src_v.at[ib], isems[0])
                pltpu.async_copy(ei_hbm.at[1, s, t], dst_v.at[ib], isems[1])

            def iwait(t):
                ib = t % 2
                pltpu.make_async_copy(
                    ei_hbm.at[0, s, t], src_v.at[ib], isems[0]).wait()
                pltpu.make_async_copy(
                    ei_hbm.at[1, s, t], dst_v.at[ib], isems[1]).wait()

            def gstart(t, j, b):
                pltpu.async_copy(g_hbm.at[src_v.at[t % 2, j]],
                                 rows_v.at[b], gsems[b])

            def gwait(t, j, b):
                pltpu.make_async_copy(g_hbm.at[src_v.at[t % 2, j]],
                                      rows_v.at[b], gsems[b]).wait()

            def scat(t, j, b):
                pltpu.sync_copy(rows_v.at[b], acc_sh.at[dst_v.at[t % 2, j]],
                                add=True)

            istart(0)
            iwait(0)
            for t in range(_NSTG):
                if t + 1 < _NSTG:
                    istart(t + 1)
                gstart(t, 0, 0)
                for j in range(_CPS):
                    b = j % 2
                    if j + 1 < _CPS:
                        gstart(t, j + 1, (j + 1) % 2)
                    gwait(t, j, b)
                    scat(t, j, b)
                if t + 1 < _NSTG:
                    iwait(t + 1)

            plsc.subcore_barrier()
            pltpu.sync_copy(acc_sh.at[pl.ds(row0, rpt)],
                            out_hbm.at[pl.ds(row0, rpt)])

            @pl.when(s == 15)
            def _():
                pltpu.sync_copy(acc_sh.at[pl.ds(tail, 16)],
                                out_hbm.at[pl.ds(tail, 16)])

        @pl.when(c == 0)
        def _():
            run(g0_hbm, out0_hbm)

        @pl.when(c == 1)
        def _():
            run(g1_hbm, out1_hbm)

    return agg


_sc_agg_128 = _make_sc_agg(128)
_sc_agg_32 = _make_sc_agg(32)


def kernel(features, norm, edge_index, W0, W1, W2):
    ei = edge_index.reshape(2, 16, _NSTG, _CPS, _K)
    z128 = jnp.zeros((624, 128), jnp.float32)
    z32 = jnp.zeros((624, 32), jnp.float32)

    g0, g1 = _tc_layer([features], norm, W0, relu_in=False)
    h0, h1 = _sc_agg_128(ei, g0, g1, z128)
    g0, g1 = _tc_layer([h0, h1], norm, W1, relu_in=True)
    h0, h1 = _sc_agg_128(ei, g0, g1, z128)
    g0, g1 = _tc_layer([h0, h1], norm, W2, relu_in=True)
    h0, h1 = _sc_agg_32(ei, g0, g1, z32)
    return _final_scale(h0, h1, norm)


# speedup vs baseline: 1.0614x; 1.0614x over previous
import functools

import jax
import jax.numpy as jnp
from jax import lax
from jax.experimental import pallas as pl
from jax.experimental.pallas import tpu as pltpu
from jax.experimental.pallas import tpu_sc as plsc

_N = 10000
_E = 160000


def _tc_layer_body(*refs, relu_in, dh, nx):
    x_refs = refs[:nx]
    norm_ref, w_ref, out0_ref, out1_ref = refs[nx:]
    if nx == 1:
        x = x_refs[0][...]
    else:
        x = jnp.concatenate([r[...] for r in x_refs], axis=1)
    nrm = norm_ref[...]
    if relu_in:
        x = jnp.maximum(x * nrm, 0.0)
    y = jnp.dot(x, w_ref[...], preferred_element_type=jnp.float32)
    y = y * nrm
    out0_ref[...] = y[:, :dh]
    out1_ref[...] = y[:, dh:]


def _tc_layer(xs, norm, w, relu_in):
    n = xs[0].shape[0]
    dout = w.shape[1]
    dh = dout // 2
    blk = 2000
    return pl.pallas_call(
        functools.partial(_tc_layer_body, relu_in=relu_in, dh=dh, nx=len(xs)),
        grid=(n // blk,),
        in_specs=[
            pl.BlockSpec((blk, x.shape[1]), lambda i: (i, 0)) for x in xs
        ] + [
            pl.BlockSpec((blk, 1), lambda i: (i, 0)),
            pl.BlockSpec(w.shape, lambda i: (0, 0)),
        ],
        out_specs=[
            pl.BlockSpec((blk, dh), lambda i: (i, 0)),
            pl.BlockSpec((blk, dh), lambda i: (i, 0)),
        ],
        out_shape=[
            jax.ShapeDtypeStruct((n, dh), jnp.float32),
            jax.ShapeDtypeStruct((n, dh), jnp.float32),
        ],
    )(*xs, norm, w)


def _scale_body(x0_ref, x1_ref, norm_ref, o_ref):
    x = jnp.concatenate([x0_ref[...], x1_ref[...]], axis=1)
    o_ref[...] = x * norm_ref[...]


def _final_scale(x0, x1, norm):
    n, dh = x0.shape
    blk = 2000
    return pl.pallas_call(
        _scale_body,
        grid=(n // blk,),
        in_specs=[
            pl.BlockSpec((blk, dh), lambda i: (i, 0)),
            pl.BlockSpec((blk, dh), lambda i: (i, 0)),
            pl.BlockSpec((blk, 1), lambda i: (i, 0)),
        ],
        out_specs=pl.BlockSpec((blk, 2 * dh), lambda i: (i, 0)),
        out_shape=jax.ShapeDtypeStruct((n, 2 * dh), jnp.float32),
    )(x0, x1, norm)


_K = 100
_NSTG = 5
_CPS = 20


def _make_sc_agg(d2):
    rpt = 624

    mesh = plsc.VectorSubcoreMesh(core_axis_name="c", subcore_axis_name="s")

    @functools.partial(
        pl.kernel,
        mesh=mesh,
        compiler_params=pltpu.CompilerParams(use_tc_tiling_on_sc=(d2 % 128 == 0)),
        out_type=[
            jax.ShapeDtypeStruct((_N, d2), jnp.float32),
            jax.ShapeDtypeStruct((_N, d2), jnp.float32),
        ],
        scratch_types=[
            pltpu.VMEM((2, _CPS, _K), jnp.int32),
            pltpu.VMEM((2, _CPS, _K), jnp.int32),
            pltpu.VMEM((2, _K, d2), jnp.float32),
            pltpu.VMEM_SHARED((_N, d2), jnp.float32),
            pltpu.SemaphoreType.DMA,
            pltpu.SemaphoreType.DMA,
            pltpu.SemaphoreType.DMA,
            pltpu.SemaphoreType.DMA,
        ],
    )
    def agg(ei_hbm, g0_hbm, g1_hbm, zero_hbm, out0_hbm, out1_hbm,
            src_v, dst_v, rows_v, acc_sh, gsem0, gsem1, isem0, isem1):
        c = lax.axis_index("c")
        s = lax.axis_index("s")
        row0 = s * rpt
        tail = 16 * rpt

        pltpu.sync_copy(zero_hbm, acc_sh.at[pl.ds(row0, rpt)])

        @pl.when(s == 15)
        def _():
            pltpu.sync_copy(zero_hbm.at[pl.ds(0, 16)],
                            acc_sh.at[pl.ds(tail, 16)])

        plsc.subcore_barrier()

        def run(g_hbm, out_hbm):
            gsems = (gsem0, gsem1)
            isems = (isem0, isem1)

            def istart(t):
                ib = t % 2
                pltpu.async_copy(ei_hbm.at[0, s, t], src_v.at[ib], isems[0])
                pltpu.async_copy(ei_hbm.at[1, s, t], dst_v.at[ib], isems[1])

            def iwait(t):
                ib = t % 2
                pltpu.make_async_copy(
                    ei_hbm.at[0, s, t], src_v.at[ib], isems[0]).wait()
                pltpu.make_async_copy(
                    ei_hbm.at[1, s, t], dst_v.at[ib], isems[1]).wait()

            def gstart(t, j, b):
                pltpu.async_copy(g_hbm.at[src_v.at[t % 2, j]],
                                 rows_v.at[b], gsems[b])

            def gwait(t, j, b):
                pltpu.make_async_copy(g_hbm.at[src_v.at[t % 2, j]],
                                      rows_v.at[b], gsems[b]).wait()

            def scat(t, j, b):
                pltpu.sync_copy(rows_v.at[b], acc_sh.at[dst_v.at[t % 2, j]],
                                add=True)

            istart(0)
            iwait(0)
            for t in range(_NSTG):
                if t + 1 < _NSTG:
                    istart(t + 1)
                gstart(t, 0, 0)

                def body(i, carry, t=t):
                    j0 = 2 * i
                    gstart(t, j0 + 1, 1)
                    gwait(t, j0, 0)
                    scat(t, j0, 0)

                    @pl.when(i < _CPS // 2 - 1)
                    def _():
                        gstart(t, j0 + 2, 0)

                    gwait(t, j0 + 1, 1)
                    scat(t, j0 + 1, 1)
                    return carry

                lax.fori_loop(0, _CPS // 2, body, 0)
                if t + 1 < _NSTG:
                    iwait(t + 1)

            plsc.subcore_barrier()
            pltpu.sync_copy(acc_sh.at[pl.ds(row0, rpt)],
                            out_hbm.at[pl.ds(row0, rpt)])

            @pl.when(s == 15)
            def _():
                pltpu.sync_copy(acc_sh.at[pl.ds(tail, 16)],
                                out_hbm.at[pl.ds(tail, 16)])

        @pl.when(c == 0)
        def _():
            run(g0_hbm, out0_hbm)

        @pl.when(c == 1)
        def _():
            run(g1_hbm, out1_hbm)

    return agg


_sc_agg_128 = _make_sc_agg(128)
_sc_agg_32 = _make_sc_agg(32)


def kernel(features, norm, edge_index, W0, W1, W2):
    ei = edge_index.reshape(2, 16, _NSTG, _CPS, _K)
    z128 = jnp.zeros((624, 128), jnp.float32)
    z32 = jnp.zeros((624, 32), jnp.float32)

    g0, g1 = _tc_layer([features], norm, W0, relu_in=False)
    h0, h1 = _sc_agg_128(ei, g0, g1, z128)
    g0, g1 = _tc_layer([h0, h1], norm, W1, relu_in=True)
    h0, h1 = _sc_agg_128(ei, g0, g1, z128)
    g0, g1 = _tc_layer([h0, h1], norm, W2, relu_in=True)
    h0, h1 = _sc_agg_32(ei, g0, g1, z32)
    return _final_scale(h0, h1, norm)


# speedup vs baseline: 1.0846x; 1.0219x over previous
import functools

import jax
import jax.numpy as jnp
from jax import lax
from jax.experimental import pallas as pl
from jax.experimental.pallas import tpu as pltpu
from jax.experimental.pallas import tpu_sc as plsc

_N = 10000
_E = 160000


def _tc_layer_body(*refs, relu_in, dh, nx):
    x_refs = refs[:nx]
    norm_ref, w_ref, out0_ref, out1_ref = refs[nx:]
    if nx == 1:
        x = x_refs[0][...]
    else:
        x = jnp.concatenate([r[...] for r in x_refs], axis=1)
    nrm = norm_ref[...]
    if relu_in:
        x = jnp.maximum(x * nrm, 0.0)
    y = jnp.dot(x, w_ref[...], preferred_element_type=jnp.float32)
    y = y * nrm
    out0_ref[...] = y[:, :dh]
    out1_ref[...] = y[:, dh:]


def _tc_layer(xs, norm, w, relu_in):
    n = xs[0].shape[0]
    dout = w.shape[1]
    dh = dout // 2
    blk = 2000
    return pl.pallas_call(
        functools.partial(_tc_layer_body, relu_in=relu_in, dh=dh, nx=len(xs)),
        grid=(n // blk,),
        in_specs=[
            pl.BlockSpec((blk, x.shape[1]), lambda i: (i, 0)) for x in xs
        ] + [
            pl.BlockSpec((blk, 1), lambda i: (i, 0)),
            pl.BlockSpec(w.shape, lambda i: (0, 0)),
        ],
        out_specs=[
            pl.BlockSpec((blk, dh), lambda i: (i, 0)),
            pl.BlockSpec((blk, dh), lambda i: (i, 0)),
        ],
        out_shape=[
            jax.ShapeDtypeStruct((n, dh), jnp.float32),
            jax.ShapeDtypeStruct((n, dh), jnp.float32),
        ],
    )(*xs, norm, w)


def _scale_body(x0_ref, x1_ref, norm_ref, o_ref):
    x = jnp.concatenate([x0_ref[...], x1_ref[...]], axis=1)
    o_ref[...] = x * norm_ref[...]


def _final_scale(x0, x1, norm):
    n, dh = x0.shape
    blk = 2000
    return pl.pallas_call(
        _scale_body,
        grid=(n // blk,),
        in_specs=[
            pl.BlockSpec((blk, dh), lambda i: (i, 0)),
            pl.BlockSpec((blk, dh), lambda i: (i, 0)),
            pl.BlockSpec((blk, 1), lambda i: (i, 0)),
        ],
        out_specs=pl.BlockSpec((blk, 2 * dh), lambda i: (i, 0)),
        out_shape=jax.ShapeDtypeStruct((n, 2 * dh), jnp.float32),
    )(x0, x1, norm)


_K = 100
_NSTG = 5
_CPS = 20


def _make_sc_agg(d2, scale_out=False):
    rpt = 624

    mesh = plsc.VectorSubcoreMesh(core_axis_name="c", subcore_axis_name="s")

    if scale_out:
        out_type = jax.ShapeDtypeStruct((_N, 2 * d2), jnp.float32)
        extra_scratch = [
            pltpu.VMEM((rpt + 16, d2), jnp.float32),
            pltpu.VMEM((rpt + 16, d2), jnp.float32),
        ]
    else:
        out_type = [
            jax.ShapeDtypeStruct((_N, d2), jnp.float32),
            jax.ShapeDtypeStruct((_N, d2), jnp.float32),
        ]
        extra_scratch = []

    def _drain_plain(out_hbm, acc_sh, s):
        row0 = s * rpt
        tail = 16 * rpt
        pltpu.sync_copy(acc_sh.at[pl.ds(row0, rpt)],
                        out_hbm.at[pl.ds(row0, rpt)])

        @pl.when(s == 15)
        def _():
            pltpu.sync_copy(acc_sh.at[pl.ds(tail, 16)],
                            out_hbm.at[pl.ds(tail, 16)])

    def _drain_scaled(out_hbm, acc_sh, stage_v, norm_v, norm_hbm, s, col0):
        row0 = s * rpt
        nrow = jnp.where(s == 15, rpt + 16, rpt)
        pltpu.sync_copy(acc_sh.at[pl.ds(row0, rpt)],
                        stage_v.at[pl.ds(0, rpt)])
        pltpu.sync_copy(norm_hbm.at[pl.ds(row0, rpt)],
                        norm_v.at[pl.ds(0, rpt)])

        @pl.when(s == 15)
        def _():
            pltpu.sync_copy(acc_sh.at[pl.ds(16 * rpt, 16)],
                            stage_v.at[pl.ds(rpt, 16)])
            pltpu.sync_copy(norm_hbm.at[pl.ds(16 * rpt, 16)],
                            norm_v.at[pl.ds(rpt, 16)])

        def srow(r, carry):
            for q in range(d2 // 16):
                stage_v[r, pl.ds(q * 16, 16)] = (
                    stage_v[r, pl.ds(q * 16, 16)]
                    * norm_v[r, pl.ds(q * 16, 16)])
            return carry

        lax.fori_loop(0, nrow, srow, 0)
        pltpu.sync_copy(stage_v.at[pl.ds(0, rpt)],
                        out_hbm.at[pl.ds(row0, rpt), pl.ds(col0, d2)])

        @pl.when(s == 15)
        def _():
            pltpu.sync_copy(stage_v.at[pl.ds(rpt, 16)],
                            out_hbm.at[pl.ds(16 * rpt, 16), pl.ds(col0, d2)])

    @functools.partial(
        pl.kernel,
        mesh=mesh,
        compiler_params=pltpu.CompilerParams(use_tc_tiling_on_sc=(d2 % 128 == 0)),
        out_type=out_type,
        scratch_types=[
            pltpu.VMEM((2, _CPS, _K), jnp.int32),
            pltpu.VMEM((2, _CPS, _K), jnp.int32),
            pltpu.VMEM((2, _K, d2), jnp.float32),
            pltpu.VMEM_SHARED((_N, d2), jnp.float32),
        ] + extra_scratch + [
            pltpu.SemaphoreType.DMA,
            pltpu.SemaphoreType.DMA,
            pltpu.SemaphoreType.DMA,
            pltpu.SemaphoreType.DMA,
        ],
    )
    def agg(*refs):
        if scale_out:
            (ei_hbm, g0_hbm, g1_hbm, zero_hbm, norm_hbm, out_hbm,
             src_v, dst_v, rows_v, acc_sh, stage_v, norm_v,
             gsem0, gsem1, isem0, isem1) = refs
        else:
            (ei_hbm, g0_hbm, g1_hbm, zero_hbm, out0_hbm, out1_hbm,
             src_v, dst_v, rows_v, acc_sh,
             gsem0, gsem1, isem0, isem1) = refs
        c = lax.axis_index("c")
        s = lax.axis_index("s")
        row0 = s * rpt

        pltpu.sync_copy(zero_hbm, acc_sh.at[pl.ds(row0, rpt)])

        @pl.when(s == 15)
        def _():
            pltpu.sync_copy(zero_hbm.at[pl.ds(0, 16)],
                            acc_sh.at[pl.ds(16 * rpt, 16)])

        plsc.subcore_barrier()

        def run(g_hbm, drain):
            gsems = (gsem0, gsem1)
            isems = (isem0, isem1)

            def istart(t):
                ib = t % 2
                pltpu.async_copy(ei_hbm.at[0, s, t], src_v.at[ib], isems[0])
                pltpu.async_copy(ei_hbm.at[1, s, t], dst_v.at[ib], isems[1])

            def iwait(t):
                ib = t % 2
                pltpu.make_async_copy(
                    ei_hbm.at[0, s, t], src_v.at[ib], isems[0]).wait()
                pltpu.make_async_copy(
                    ei_hbm.at[1, s, t], dst_v.at[ib], isems[1]).wait()

            def gstart(t, j, b):
                pltpu.async_copy(g_hbm.at[src_v.at[t % 2, j]],
                                 rows_v.at[b], gsems[b])

            def gwait(t, j, b):
                pltpu.make_async_copy(g_hbm.at[src_v.at[t % 2, j]],
                                      rows_v.at[b], gsems[b]).wait()

            def scat(t, j, b):
                pltpu.sync_copy(rows_v.at[b], acc_sh.at[dst_v.at[t % 2, j]],
                                add=True)

            istart(0)
            iwait(0)
            for t in range(_NSTG):
                if t + 1 < _NSTG:
                    istart(t + 1)
                gstart(t, 0, 0)

                def body(i, carry, t=t):
                    j0 = 2 * i
                    gstart(t, j0 + 1, 1)
                    gwait(t, j0, 0)
                    scat(t, j0, 0)

                    @pl.when(i < _CPS // 2 - 1)
                    def _():
                        gstart(t, j0 + 2, 0)

                    gwait(t, j0 + 1, 1)
                    scat(t, j0 + 1, 1)
                    return carry

                lax.fori_loop(0, _CPS // 2, body, 0)
                if t + 1 < _NSTG:
                    iwait(t + 1)

            plsc.subcore_barrier()
            drain()

        if scale_out:
            @pl.when(c == 0)
            def _():
                run(g0_hbm, lambda: _drain_scaled(
                    out_hbm, acc_sh, stage_v, norm_v, norm_hbm, s, 0))

            @pl.when(c == 1)
            def _():
                run(g1_hbm, lambda: _drain_scaled(
                    out_hbm, acc_sh, stage_v, norm_v, norm_hbm, s, d2))
        else:
            @pl.when(c == 0)
            def _():
                run(g0_hbm, lambda: _drain_plain(out0_hbm, acc_sh, s))

            @pl.when(c == 1)
            def _():
                run(g1_hbm, lambda: _drain_plain(out1_hbm, acc_sh, s))

    return agg


_sc_agg_128 = _make_sc_agg(128)
_sc_agg_32s = _make_sc_agg(32, scale_out=True)


def kernel(features, norm, edge_index, W0, W1, W2):
    ei = edge_index.reshape(2, 16, _NSTG, _CPS, _K)
    z128 = jnp.zeros((624, 128), jnp.float32)
    z32 = jnp.zeros((624, 32), jnp.float32)

    g0, g1 = _tc_layer([features], norm, W0, relu_in=False)
    h0, h1 = _sc_agg_128(ei, g0, g1, z128)
    g0, g1 = _tc_layer([h0, h1], norm, W1, relu_in=True)
    h0, h1 = _sc_agg_128(ei, g0, g1, z128)
    g0, g1 = _tc_layer([h0, h1], norm, W2, relu_in=True)
    nrep = jnp.broadcast_to(norm, (_N, 32))
    return _sc_agg_32s(ei, g0, g1, z32, nrep)


# speedup vs baseline: 1.0932x; 1.0079x over previous
import functools

import jax
import jax.numpy as jnp
from jax import lax
from jax.experimental import pallas as pl
from jax.experimental.pallas import tpu as pltpu
from jax.experimental.pallas import tpu_sc as plsc

_N = 10000
_E = 160000


def _tc_layer_body(*refs, relu_in, dh, nx):
    x_refs = refs[:nx]
    norm_ref, w_ref, out0_ref, out1_ref = refs[nx:]
    if nx == 1:
        x = x_refs[0][...]
    else:
        x = jnp.concatenate([r[...] for r in x_refs], axis=1)
    nrm = norm_ref[...]
    if relu_in:
        x = jnp.maximum(x * nrm, 0.0)
    y = jnp.dot(x, w_ref[...], preferred_element_type=jnp.float32)
    y = y * nrm
    out0_ref[...] = y[:, :dh]
    out1_ref[...] = y[:, dh:]


def _tc_layer(xs, norm, w, relu_in):
    n = xs[0].shape[0]
    dout = w.shape[1]
    dh = dout // 2
    blk = 2000
    return pl.pallas_call(
        functools.partial(_tc_layer_body, relu_in=relu_in, dh=dh, nx=len(xs)),
        grid=(n // blk,),
        in_specs=[
            pl.BlockSpec((blk, x.shape[1]), lambda i: (i, 0)) for x in xs
        ] + [
            pl.BlockSpec((blk, 1), lambda i: (i, 0)),
            pl.BlockSpec(w.shape, lambda i: (0, 0)),
        ],
        out_specs=[
            pl.BlockSpec((blk, dh), lambda i: (i, 0)),
            pl.BlockSpec((blk, dh), lambda i: (i, 0)),
        ],
        out_shape=[
            jax.ShapeDtypeStruct((n, dh), jnp.float32),
            jax.ShapeDtypeStruct((n, dh), jnp.float32),
        ],
    )(*xs, norm, w)


_K = 100
_NSTG = 5
_CPS = 20


def _make_sc_agg(d2, scale_out=False):
    rpt = 624

    mesh = plsc.VectorSubcoreMesh(core_axis_name="c", subcore_axis_name="s")

    if scale_out:
        out_type = jax.ShapeDtypeStruct((_N, 2 * d2), jnp.float32)
        extra_scratch = [
            pltpu.VMEM((rpt + 16, d2), jnp.float32),
            pltpu.VMEM((rpt + 16, d2), jnp.float32),
        ]
    else:
        out_type = [
            jax.ShapeDtypeStruct((_N, d2), jnp.float32),
            jax.ShapeDtypeStruct((_N, d2), jnp.float32),
        ]
        extra_scratch = []

    def _drain_plain(out_hbm, acc_sh, s):
        row0 = s * rpt
        tail = 16 * rpt
        pltpu.sync_copy(acc_sh.at[pl.ds(row0, rpt)],
                        out_hbm.at[pl.ds(row0, rpt)])

        @pl.when(s == 15)
        def _():
            pltpu.sync_copy(acc_sh.at[pl.ds(tail, 16)],
                            out_hbm.at[pl.ds(tail, 16)])

    def _drain_scaled(out_hbm, acc_sh, stage_v, norm_v, norm_hbm, s, col0):
        row0 = s * rpt
        nrow = jnp.where(s == 15, rpt + 16, rpt)
        pltpu.sync_copy(acc_sh.at[pl.ds(row0, rpt)],
                        stage_v.at[pl.ds(0, rpt)])
        pltpu.sync_copy(norm_hbm.at[pl.ds(row0, rpt)],
                        norm_v.at[pl.ds(0, rpt)])

        @pl.when(s == 15)
        def _():
            pltpu.sync_copy(acc_sh.at[pl.ds(16 * rpt, 16)],
                            stage_v.at[pl.ds(rpt, 16)])
            pltpu.sync_copy(norm_hbm.at[pl.ds(16 * rpt, 16)],
                            norm_v.at[pl.ds(rpt, 16)])

        def srow(r, carry):
            for q in range(d2 // 16):
                stage_v[r, pl.ds(q * 16, 16)] = (
                    stage_v[r, pl.ds(q * 16, 16)]
                    * norm_v[r, pl.ds(q * 16, 16)])
            return carry

        lax.fori_loop(0, nrow, srow, 0)
        pltpu.sync_copy(stage_v.at[pl.ds(0, rpt)],
                        out_hbm.at[pl.ds(row0, rpt), pl.ds(col0, d2)])

        @pl.when(s == 15)
        def _():
            pltpu.sync_copy(stage_v.at[pl.ds(rpt, 16)],
                            out_hbm.at[pl.ds(16 * rpt, 16), pl.ds(col0, d2)])

    @functools.partial(
        pl.kernel,
        mesh=mesh,
        compiler_params=pltpu.CompilerParams(use_tc_tiling_on_sc=(d2 % 128 == 0)),
        out_type=out_type,
        scratch_types=[
            pltpu.VMEM((2, _CPS, _K), jnp.int32),
            pltpu.VMEM((2, _CPS, _K), jnp.int32),
            pltpu.VMEM((2, _K, d2), jnp.float32),
            pltpu.VMEM_SHARED((_N, d2), jnp.float32),
        ] + extra_scratch + [
            pltpu.SemaphoreType.DMA,
            pltpu.SemaphoreType.DMA,
            pltpu.SemaphoreType.DMA,
            pltpu.SemaphoreType.DMA,
        ],
    )
    def agg(*refs):
        if scale_out:
            (ei_hbm, g0_hbm, g1_hbm, zero_hbm, norm_hbm, out_hbm,
             src_v, dst_v, rows_v, acc_sh, stage_v, norm_v,
             gsem0, gsem1, isem0, isem1) = refs
        else:
            (ei_hbm, g0_hbm, g1_hbm, zero_hbm, out0_hbm, out1_hbm,
             src_v, dst_v, rows_v, acc_sh,
             gsem0, gsem1, isem0, isem1) = refs
        c = lax.axis_index("c")
        s = lax.axis_index("s")
        row0 = s * rpt

        def run(g_hbm, drain):
            gsems = (gsem0, gsem1)
            isems = (isem0, isem1)

            def istart(t):
                ib = t % 2
                pltpu.async_copy(ei_hbm.at[0, s, t], src_v.at[ib], isems[0])
                pltpu.async_copy(ei_hbm.at[1, s, t], dst_v.at[ib], isems[1])

            def iwait(t):
                ib = t % 2
                pltpu.make_async_copy(
                    ei_hbm.at[0, s, t], src_v.at[ib], isems[0]).wait()
                pltpu.make_async_copy(
                    ei_hbm.at[1, s, t], dst_v.at[ib], isems[1]).wait()

            def gstart(t, j, b):
                pltpu.async_copy(g_hbm.at[src_v.at[t % 2, j]],
                                 rows_v.at[b], gsems[b])

            def gwait(t, j, b):
                pltpu.make_async_copy(g_hbm.at[src_v.at[t % 2, j]],
                                      rows_v.at[b], gsems[b]).wait()

            def scat(t, j, b):
                pltpu.sync_copy(rows_v.at[b], acc_sh.at[dst_v.at[t % 2, j]],
                                add=True)

            istart(0)
            iwait(0)
            gstart(0, 0, 0)

            pltpu.sync_copy(zero_hbm, acc_sh.at[pl.ds(row0, rpt)])

            @pl.when(s == 15)
            def _():
                pltpu.sync_copy(zero_hbm.at[pl.ds(0, 16)],
                                acc_sh.at[pl.ds(16 * rpt, 16)])

            plsc.subcore_barrier()

            for t in range(_NSTG):
                if t + 1 < _NSTG:
                    istart(t + 1)
                if t > 0:
                    gstart(t, 0, 0)

                def body(i, carry, t=t):
                    j0 = 2 * i
                    gstart(t, j0 + 1, 1)
                    gwait(t, j0, 0)
                    scat(t, j0, 0)

                    @pl.when(i < _CPS // 2 - 1)
                    def _():
                        gstart(t, j0 + 2, 0)

                    gwait(t, j0 + 1, 1)
                    scat(t, j0 + 1, 1)
                    return carry

                lax.fori_loop(0, _CPS // 2, body, 0)
                if t + 1 < _NSTG:
                    iwait(t + 1)

            plsc.subcore_barrier()
            drain()

        if scale_out:
            @pl.when(c == 0)
            def _():
                run(g0_hbm, lambda: _drain_scaled(
                    out_hbm, acc_sh, stage_v, norm_v, norm_hbm, s, 0))

            @pl.when(c == 1)
            def _():
                run(g1_hbm, lambda: _drain_scaled(
                    out_hbm, acc_sh, stage_v, norm_v, norm_hbm, s, d2))
        else:
            @pl.when(c == 0)
            def _():
                run(g0_hbm, lambda: _drain_plain(out0_hbm, acc_sh, s))

            @pl.when(c == 1)
            def _():
                run(g1_hbm, lambda: _drain_plain(out1_hbm, acc_sh, s))

    return agg


_sc_agg_128 = _make_sc_agg(128)
_sc_agg_32s = _make_sc_agg(32, scale_out=True)


def kernel(features, norm, edge_index, W0, W1, W2):
    ei = edge_index.reshape(2, 16, _NSTG, _CPS, _K)
    z128 = jnp.zeros((624, 128), jnp.float32)
    z32 = jnp.zeros((624, 32), jnp.float32)

    g0, g1 = _tc_layer([features], norm, W0, relu_in=False)
    h0, h1 = _sc_agg_128(ei, g0, g1, z128)
    g0, g1 = _tc_layer([h0, h1], norm, W1, relu_in=True)
    h0, h1 = _sc_agg_128(ei, g0, g1, z128)
    g0, g1 = _tc_layer([h0, h1], norm, W2, relu_in=True)
    nrep = jnp.broadcast_to(norm, (_N, 32))
    return _sc_agg_32s(ei, g0, g1, z32, nrep)
